# Initial kernel scaffold; baseline (speedup 1.0000x reference)
import jax
import jax.numpy as jnp
from jax.experimental import pallas as pl


def kernel(phrase_ids, morph_ids, word_table, morph_table):
    raise NotImplementedError("write your pallas kernel here")


# speedup vs baseline: 2.2978x; 2.2978x over previous
import functools

import jax
import jax.numpy as jnp
from jax import lax
from jax.experimental import pallas as pl
from jax.experimental.pallas import tpu as pltpu
from jax.experimental.pallas import tpu_sc as plsc

NC = 2
NS = 16
NW = NC * NS
L = 16

D = 64
C = 512


@functools.partial(jax.jit, static_argnames=("n_rows",))
def _fused_lookup(pid, mid, word_table, morph_table, *, n_rows):
    per_w = n_rows // NW
    n_chunks = per_w // C

    mesh = plsc.VectorSubcoreMesh(core_axis_name="c", subcore_axis_name="s")

    @functools.partial(
        pl.kernel,
        out_type=jax.ShapeDtypeStruct((n_rows, D), jnp.float32),
        mesh=mesh,
        compiler_params=pltpu.CompilerParams(use_tc_tiling_on_sc=False),
        scratch_types=[
            pltpu.VMEM((C,), jnp.int32),
            pltpu.VMEM((C,), jnp.int32),
            pltpu.VMEM((C, D), jnp.float32),
            pltpu.VMEM((C, D), jnp.float32),
            pltpu.SemaphoreType.DMA,
            pltpu.SemaphoreType.DMA,
        ],
    )
    def body(pid_hbm, mid_hbm, word_hbm, morph_hbm, out_hbm,
             idxw, idxm, roww, rowm, semw, semm):
        wid = lax.axis_index("s") * NC + lax.axis_index("c")
        base = wid * per_w

        def chunk(g, carry):
            off = base + g * C
            pltpu.sync_copy(pid_hbm.at[pl.ds(off, C)], idxw)
            pltpu.sync_copy(mid_hbm.at[pl.ds(off, C)], idxm)
            cw = pltpu.async_copy(word_hbm.at[idxw], roww, semw)
            cm = pltpu.async_copy(morph_hbm.at[idxm], rowm, semm)
            cw.wait()
            cm.wait()

            def row(i, carry2):
                for c in range(D // L):
                    a = roww[i, pl.ds(c * L, L)]
                    b = rowm[i, pl.ds(c * L, L)]
                    roww[i, pl.ds(c * L, L)] = (a + b) * 0.5
                return carry2

            lax.fori_loop(0, C, row, 0, unroll=False)
            pltpu.sync_copy(roww, out_hbm.at[pl.ds(off, C)])
            return carry

        lax.fori_loop(0, n_chunks, chunk, 0, unroll=False)

    return body(pid, mid, word_table, morph_table)


def kernel(phrase_ids, morph_ids, word_table, morph_table):
    batch, seq = phrase_ids.shape
    n_rows = batch * seq
    out = _fused_lookup(
        phrase_ids.reshape(n_rows),
        morph_ids.reshape(n_rows),
        word_table,
        morph_table,
        n_rows=n_rows,
    )
    return out.reshape(batch, seq, D)
